# baseline (speedup 1.0000x reference)
import functools

import jax
import jax.numpy as jnp
from jax import lax
from jax.experimental import pallas as pl
from jax.experimental.pallas import tpu as pltpu
from jax.experimental.pallas import tpu_sc as plsc

BATCH, HIST = 4096, 50
N = BATCH * HIST
LANES = 128
NROWS = N // LANES
D = 64
OUT_D = 3 * D
TRES = 24.0
TSCALE = 10.0
V1, V2, V3 = 366, 24, 100000

NC, NS = 2, 16
NW = NC * NS
RPW = NROWS // NW
SLOTS = 4

GRID = 25
W3B = V3 // GRID
IDXB = NROWS // GRID


def _prep_body(idx_ref, w1_ref, w2_ref, w3_ref,
               i1_ref, i2_ref, i3_ref, o1_ref, o2_ref, o3_ref):
    j = pl.program_id(0)
    x = idx_ref[...]
    t1 = (x * TRES).astype(jnp.int32)
    i1_ref[...] = lax.rem(lax.div(t1, 24), 366)
    i2_ref[...] = lax.rem(t1, 24)
    i3_ref[...] = (x / TSCALE).astype(jnp.int32)

    def scaled(w):
        nrm = jnp.sqrt(jnp.sum(w * w, axis=-1, keepdims=True))
        return w * jnp.where(nrm > 1.0, 1.0 / (nrm + 1e-7), 1.0)

    o3_ref[...] = scaled(w3_ref[...])

    @pl.when(j == 0)
    def _():
        o1_ref[...] = scaled(w1_ref[...])
        o2_ref[...] = scaled(w2_ref[...])


_prep = pl.pallas_call(
    _prep_body,
    grid=(GRID,),
    in_specs=[
        pl.BlockSpec((IDXB, LANES), lambda j: (j, 0)),
        pl.BlockSpec((V1, D), lambda j: (0, 0)),
        pl.BlockSpec((V2, D), lambda j: (0, 0)),
        pl.BlockSpec((W3B, D), lambda j: (j, 0)),
    ],
    out_specs=[
        pl.BlockSpec((IDXB, LANES), lambda j: (j, 0)),
        pl.BlockSpec((IDXB, LANES), lambda j: (j, 0)),
        pl.BlockSpec((IDXB, LANES), lambda j: (j, 0)),
        pl.BlockSpec((V1, D), lambda j: (0, 0)),
        pl.BlockSpec((V2, D), lambda j: (0, 0)),
        pl.BlockSpec((W3B, D), lambda j: (j, 0)),
    ],
    out_shape=[
        jax.ShapeDtypeStruct((NROWS, LANES), jnp.int32),
        jax.ShapeDtypeStruct((NROWS, LANES), jnp.int32),
        jax.ShapeDtypeStruct((NROWS, LANES), jnp.int32),
        jax.ShapeDtypeStruct((V1, D), jnp.float32),
        jax.ShapeDtypeStruct((V2, D), jnp.float32),
        jax.ShapeDtypeStruct((V3, D), jnp.float32),
    ],
)


@functools.partial(
    pl.kernel,
    out_type=jax.ShapeDtypeStruct((N, OUT_D), jnp.float32),
    mesh=plsc.VectorSubcoreMesh(core_axis_name="c", subcore_axis_name="s",
                                num_cores=NC, num_subcores=NS),
    compiler_params=pltpu.CompilerParams(use_tc_tiling_on_sc=False),
    scratch_types=[
        pltpu.VMEM((RPW, LANES), jnp.int32),
        pltpu.VMEM((RPW, LANES), jnp.int32),
        pltpu.VMEM((RPW, LANES), jnp.int32),
        pltpu.VMEM((SLOTS, 3, LANES, D), jnp.float32),
        pltpu.SemaphoreType.DMA,
        pltpu.SemaphoreType.DMA,
        pltpu.SemaphoreType.DMA,
        pltpu.SemaphoreType.DMA,
        pltpu.SemaphoreType.DMA,
        pltpu.SemaphoreType.DMA,
    ],
)
def _gather(i1_hbm, i2_hbm, i3_hbm, w1_hbm, w2_hbm, w3_hbm, out_hbm,
            iv1, iv2, iv3, bufs, g1, g2, g3, s1, s2, s3):
    wid = lax.axis_index("s") * NC + lax.axis_index("c")
    rbase = wid * RPW

    pltpu.sync_copy(i1_hbm.at[wid], iv1)
    pltpu.sync_copy(i2_hbm.at[wid], iv2)
    pltpu.sync_copy(i3_hbm.at[wid], iv3)

    gsems = (g1, g2, g3)
    wsems = (s1, s2, s3)
    tabs = (w1_hbm, w2_hbm, w3_hbm)
    ivs = (iv1, iv2, iv3)

    def gathers(j, slot):
        return tuple(
            pltpu.make_async_copy(tabs[t].at[ivs[t].at[j]], bufs.at[slot, t],
                                  gsems[t])
            for t in range(3)
        )

    def writes(j, slot):
        ob = (rbase + j) * LANES
        return tuple(
            pltpu.make_async_copy(
                bufs.at[slot, t],
                out_hbm.at[pl.ds(ob, LANES), pl.ds(t * D, D)],
                wsems[t])
            for t in range(3)
        )

    for p in range(SLOTS - 1):
        for c in gathers(p, p):
            c.start()

    def body(j, carry):
        slot = lax.rem(j, SLOTS)
        nslot = lax.rem(j + SLOTS - 1, SLOTS)
        @pl.when(jnp.logical_and(j >= 1, j + SLOTS - 1 < RPW))
        def _():
            for c in writes(0, 0):
                c.wait()

        @pl.when(j + SLOTS - 1 < RPW)
        def _():
            for c in gathers(j + SLOTS - 1, nslot):
                c.start()

        for c in gathers(j, slot):
            c.wait()
        for c in writes(j, slot):
            c.start()
        return carry

    lax.fori_loop(0, RPW, body, 0)

    for _ in range(SLOTS):
        for c in writes(0, 0):
            c.wait()


def kernel(idx, W1, W2, W3):
    idxr = idx.reshape(NROWS, LANES)
    i1, i2, i3, w1s, w2s, w3s = _prep(idxr, W1, W2, W3)
    i1 = i1.reshape(NW, RPW, LANES)
    i2 = i2.reshape(NW, RPW, LANES)
    i3 = i3.reshape(NW, RPW, LANES)
    out = _gather(i1, i2, i3, w1s, w2s, w3s)
    return out.reshape(BATCH, HIST, OUT_D)


# speedup vs baseline: 1.4753x; 1.4753x over previous
import functools

import jax
import jax.numpy as jnp
from jax import lax
from jax.experimental import pallas as pl
from jax.experimental.pallas import tpu as pltpu
from jax.experimental.pallas import tpu_sc as plsc

BATCH, HIST = 4096, 50
D = 64
OUT_D = 3 * D
TRES = 24.0
TSCALE = 10.0
V1, V2, V3 = 366, 24, 100000

NC, NS = 2, 16
NW = NC * NS
BPW = BATCH // NW
SLOTS = 5
AHEAD = 3

IGRID = 32
IBLK = BATCH // IGRID
RGRID = 25
W3B = V3 // RGRID


HISTP = 64


def _prep_idx_body(idx_ref, i1_ref, i2_ref, i3_ref):
    x = idx_ref[...]
    t1 = (x * TRES).astype(jnp.int32)
    pad = jnp.zeros((IBLK, HISTP - HIST), jnp.int32)
    i1_ref[...] = jnp.concatenate(
        [lax.rem(lax.div(t1, 24), 366), pad], axis=1)
    i2_ref[...] = jnp.concatenate([lax.rem(t1, 24), pad], axis=1)
    i3_ref[...] = (x / TSCALE).astype(jnp.int32)


_prep_idx = pl.pallas_call(
    _prep_idx_body,
    grid=(IGRID,),
    in_specs=[pl.BlockSpec((IBLK, HIST), lambda j: (j, 0))],
    out_specs=[
        pl.BlockSpec((IBLK, HISTP), lambda j: (j, 0)),
        pl.BlockSpec((IBLK, HISTP), lambda j: (j, 0)),
        pl.BlockSpec((IBLK, HIST), lambda j: (j, 0)),
    ],
    out_shape=[
        jax.ShapeDtypeStruct((BATCH, HISTP), jnp.int32),
        jax.ShapeDtypeStruct((BATCH, HISTP), jnp.int32),
        jax.ShapeDtypeStruct((BATCH, HIST), jnp.int32),
    ],
)


def _prep_renorm_body(w1_ref, w2_ref, w3_ref, o1_ref, o2_ref, o3_ref):
    j = pl.program_id(0)

    def scaled(w):
        nrm = jnp.sqrt(jnp.sum(w * w, axis=-1, keepdims=True))
        return w * jnp.where(nrm > 1.0, 1.0 / (nrm + 1e-7), 1.0)

    o3_ref[...] = scaled(w3_ref[...])

    @pl.when(j == 0)
    def _():
        o1_ref[...] = scaled(w1_ref[...])
        o2_ref[...] = scaled(w2_ref[...])


_prep_renorm = pl.pallas_call(
    _prep_renorm_body,
    grid=(RGRID,),
    in_specs=[
        pl.BlockSpec((V1, D), lambda j: (0, 0)),
        pl.BlockSpec((V2, D), lambda j: (0, 0)),
        pl.BlockSpec((W3B, D), lambda j: (j, 0)),
    ],
    out_specs=[
        pl.BlockSpec((V1, D), lambda j: (0, 0)),
        pl.BlockSpec((V2, D), lambda j: (0, 0)),
        pl.BlockSpec((W3B, D), lambda j: (j, 0)),
    ],
    out_shape=[
        jax.ShapeDtypeStruct((V1, D), jnp.float32),
        jax.ShapeDtypeStruct((V2, D), jnp.float32),
        jax.ShapeDtypeStruct((V3, D), jnp.float32),
    ],
)


@functools.partial(
    pl.kernel,
    out_type=jax.ShapeDtypeStruct((BATCH, HIST, OUT_D), jnp.float32),
    mesh=plsc.VectorSubcoreMesh(core_axis_name="c", subcore_axis_name="s",
                                num_cores=NC, num_subcores=NS),
    compiler_params=pltpu.CompilerParams(use_tc_tiling_on_sc=False),
    scratch_types=[
        pltpu.VMEM((BPW, HISTP), jnp.int32),
        pltpu.VMEM((BPW, HISTP), jnp.int32),
        pltpu.VMEM((BPW, HIST), jnp.int32),
        pltpu.VMEM((V1, D), jnp.float32),
        pltpu.VMEM((V2, D), jnp.float32),
        pltpu.VMEM((SLOTS, HIST, OUT_D), jnp.float32),
        pltpu.VMEM((SLOTS, HIST, D), jnp.float32),
        pltpu.SemaphoreType.DMA,
        pltpu.SemaphoreType.DMA,
    ],
)
def _gather(i1_hbm, i2_hbm, i3_hbm, w1_hbm, w2_hbm, w3_hbm, out_hbm,
            iv1, iv2, iv3, w1v, w2v, obuf, e3buf, gsem, wsem):
    wid = lax.axis_index("s") * NC + lax.axis_index("c")
    bbase = wid * BPW

    pltpu.sync_copy(w1_hbm, w1v)
    pltpu.sync_copy(w2_hbm, w2v)
    pltpu.sync_copy(i1_hbm.at[pl.ds(bbase, BPW)], iv1)
    pltpu.sync_copy(i2_hbm.at[pl.ds(bbase, BPW)], iv2)
    pltpu.sync_copy(i3_hbm.at[pl.ds(bbase, BPW)], iv3)

    def e3_gather(j, slot):
        return pltpu.make_async_copy(w3_hbm.at[iv3.at[j]], e3buf.at[slot],
                                     gsem)

    def write(j, slot):
        return pltpu.make_async_copy(obuf.at[slot], out_hbm.at[bbase + j],
                                     wsem)

    for p in range(AHEAD):
        e3_gather(p, p).start()

    def body(j, carry):
        slot = lax.rem(j, SLOTS)
        nslot = lax.rem(j + AHEAD, SLOTS)

        @pl.when(jnp.logical_and(j + AHEAD >= SLOTS, j + AHEAD < BPW))
        def _():
            write(0, 0).wait()

        @pl.when(j + AHEAD < BPW)
        def _():
            e3_gather(j + AHEAD, nslot).start()

        e3_gather(j, slot).wait()

        for g in range(4):
            av1 = iv1[j, pl.ds(g * 16, 16)]
            av2 = iv2[j, pl.ds(g * 16, 16)]
            for lane in range(16 if g < 3 else HIST - 48):
                r = g * 16 + lane
                a = av1[lane]
                b = av2[lane]
                for q in range(4):
                    obuf[slot, r, pl.ds(q * 16, 16)] = (
                        w1v[a, pl.ds(q * 16, 16)])
                for q in range(4):
                    obuf[slot, r, pl.ds(D + q * 16, 16)] = (
                        w2v[b, pl.ds(q * 16, 16)])
                for q in range(4):
                    obuf[slot, r, pl.ds(2 * D + q * 16, 16)] = (
                        e3buf[slot, r, pl.ds(q * 16, 16)])

        write(j, slot).start()
        return carry

    lax.fori_loop(0, BPW, body, 0)

    for _ in range(SLOTS):
        write(0, 0).wait()


def kernel(idx, W1, W2, W3):
    idx2 = idx.reshape(BATCH, HIST)
    i1, i2, i3 = _prep_idx(idx2)
    w1s, w2s, w3s = _prep_renorm(W1, W2, W3)
    return _gather(i1, i2, i3, w1s, w2s, w3s)


# speedup vs baseline: 1.5562x; 1.0548x over previous
import functools

import jax
import jax.numpy as jnp
from jax import lax
from jax.experimental import pallas as pl
from jax.experimental.pallas import tpu as pltpu
from jax.experimental.pallas import tpu_sc as plsc

BATCH, HIST = 4096, 50
D = 64
OUT_D = 3 * D
TRES = 24.0
TSCALE = 10.0
V1, V2, V3 = 366, 24, 100000
V1H, V2H = V1 // 2, V2 // 2

NC, NS = 2, 16
NW = NC * NS
BPW = BATCH // NW
SLOTS = 3
AHEAD = 2

IGRID = 32
IBLK = BATCH // IGRID
RGRID = 25
W3B = V3 // RGRID


def _prep_idx_body(idx_ref, i12_ref, i3_ref):
    x = idx_ref[...]
    t1 = (x * TRES).astype(jnp.int32)
    e1 = lax.rem(lax.div(t1, 24), 366)
    e2 = lax.rem(t1, 24)
    e3 = (x / TSCALE).astype(jnp.int32)
    z = jnp.zeros((IBLK, D - HIST), jnp.int32)
    i12_ref[...] = jnp.concatenate([e1, z, e2, z], axis=1)
    i3_ref[...] = jnp.concatenate(
        [e3, jnp.zeros((IBLK, 2 * D - HIST), jnp.int32)], axis=1)


_prep_idx = pl.pallas_call(
    _prep_idx_body,
    grid=(IGRID,),
    in_specs=[pl.BlockSpec((IBLK, HIST), lambda j: (j, 0))],
    out_specs=[
        pl.BlockSpec((IBLK, 2 * D), lambda j: (j, 0)),
        pl.BlockSpec((IBLK, 2 * D), lambda j: (j, 0)),
    ],
    out_shape=[
        jax.ShapeDtypeStruct((BATCH, 2 * D), jnp.int32),
        jax.ShapeDtypeStruct((BATCH, 2 * D), jnp.int32),
    ],
)


def _scaled_rows(w):
    nrm = jnp.sqrt(jnp.sum(w * w, axis=-1, keepdims=True))
    return w * jnp.where(nrm > 1.0, 1.0 / (nrm + 1e-7), 1.0)


def _prep_renorm_body(w1_ref, w2_ref, w3_ref, o1_ref, o2_ref, o3_ref):
    j = pl.program_id(0)

    s3 = _scaled_rows(w3_ref[...])
    o3_ref[...] = jnp.concatenate([s3, jnp.zeros((W3B, D), jnp.float32)],
                                  axis=1)

    @pl.when(j == 0)
    def _():
        s1 = _scaled_rows(w1_ref[...])
        o1_ref[...] = jnp.concatenate([s1[:V1H], s1[V1H:]], axis=1)
        s2 = _scaled_rows(w2_ref[...])
        o2_ref[...] = jnp.concatenate([s2[:V2H], s2[V2H:]], axis=1)


_prep_renorm = pl.pallas_call(
    _prep_renorm_body,
    grid=(RGRID,),
    in_specs=[
        pl.BlockSpec((V1, D), lambda j: (0, 0)),
        pl.BlockSpec((V2, D), lambda j: (0, 0)),
        pl.BlockSpec((W3B, D), lambda j: (j, 0)),
    ],
    out_specs=[
        pl.BlockSpec((V1H, 2 * D), lambda j: (0, 0)),
        pl.BlockSpec((V2H, 2 * D), lambda j: (0, 0)),
        pl.BlockSpec((W3B, 2 * D), lambda j: (j, 0)),
    ],
    out_shape=[
        jax.ShapeDtypeStruct((V1H, 2 * D), jnp.float32),
        jax.ShapeDtypeStruct((V2H, 2 * D), jnp.float32),
        jax.ShapeDtypeStruct((V3, 2 * D), jnp.float32),
    ],
)


@functools.partial(
    pl.kernel,
    out_type=jax.ShapeDtypeStruct((BATCH, HIST, OUT_D), jnp.float32),
    mesh=plsc.VectorSubcoreMesh(core_axis_name="c", subcore_axis_name="s",
                                num_cores=NC, num_subcores=NS),
    compiler_params=pltpu.CompilerParams(use_tc_tiling_on_sc=True),
    scratch_types=[
        pltpu.VMEM((BPW, 2 * D), jnp.int32),
        pltpu.VMEM((BPW, 2 * D), jnp.int32),
        pltpu.VMEM((V1H, 2 * D), jnp.float32),
        pltpu.VMEM((V2H, 2 * D), jnp.float32),
        pltpu.VMEM((SLOTS, HIST, OUT_D), jnp.float32),
        pltpu.VMEM((SLOTS, HIST, 2 * D), jnp.float32),
        pltpu.SemaphoreType.DMA,
        pltpu.SemaphoreType.DMA,
    ],
)
def _gather(i12_hbm, i3_hbm, w1_hbm, w2_hbm, w3_hbm, out_hbm,
            iv12, iv3, w1v, w2v, obuf, e3buf, gsem, wsem):
    wid = lax.axis_index("s") * NC + lax.axis_index("c")
    bbase = wid * BPW

    pltpu.sync_copy(w1_hbm, w1v)
    pltpu.sync_copy(w2_hbm, w2v)
    pltpu.sync_copy(i12_hbm.at[pl.ds(bbase, BPW)], iv12)
    pltpu.sync_copy(i3_hbm.at[pl.ds(bbase, BPW)], iv3)

    def e3_gather(j, slot):
        idx = iv3.at[j, pl.ds(0, HIST)]
        return pltpu.make_async_copy(w3_hbm.at[idx], e3buf.at[slot], gsem)

    def write(j, slot):
        return pltpu.make_async_copy(obuf.at[slot], out_hbm.at[bbase + j],
                                     wsem)

    for p in range(AHEAD):
        e3_gather(p, p).start()

    def body(j, carry):
        slot = lax.rem(j, SLOTS)
        nslot = lax.rem(j + AHEAD, SLOTS)

        @pl.when(jnp.logical_and(j + AHEAD >= SLOTS, j + AHEAD < BPW))
        def _():
            write(0, 0).wait()

        @pl.when(j + AHEAD < BPW)
        def _():
            e3_gather(j + AHEAD, nslot).start()

        e3_gather(j, slot).wait()

        for g in range(4):
            av1 = iv12[j, pl.ds(g * 16, 16)]
            av2 = iv12[j, pl.ds(D + g * 16, 16)]
            for lane in range(16 if g < 3 else HIST - 48):
                r = g * 16 + lane
                a = av1[lane]
                b = av2[lane]
                ah = (a >= V1H).astype(jnp.int32)
                bh = (b >= V2H).astype(jnp.int32)
                ar = a - V1H * ah
                br = b - V2H * bh
                ao = D * ah
                bo = D * bh
                for q in range(4):
                    obuf[slot, r, pl.ds(q * 16, 16)] = (
                        w1v[ar, pl.ds(ao + q * 16, 16)])
                for q in range(4):
                    obuf[slot, r, pl.ds(D + q * 16, 16)] = (
                        w2v[br, pl.ds(bo + q * 16, 16)])
                for q in range(4):
                    obuf[slot, r, pl.ds(2 * D + q * 16, 16)] = (
                        e3buf[slot, r, pl.ds(q * 16, 16)])

        write(j, slot).start()
        return carry

    lax.fori_loop(0, BPW, body, 0)

    for _ in range(SLOTS):
        write(0, 0).wait()


def kernel(idx, W1, W2, W3):
    idx2 = idx.reshape(BATCH, HIST)
    i12, i3 = _prep_idx(idx2)
    w1s, w2s, w3s = _prep_renorm(W1, W2, W3)
    return _gather(i12, i3, w1s, w2s, w3s)
